# baseline (speedup 1.0000x reference)
import functools

import jax
import jax.numpy as jnp
from jax import lax
from jax.experimental import pallas as pl
from jax.experimental.pallas import tpu as pltpu
from jax.experimental.pallas import tpu_sc as plsc

NC = 2
NS = 16
NW = NC * NS
CHUNK = 32


NBUF = 2


def _make_sc_kernel(seq_len, b_sz, emb, dtype):
    rows_per_w = seq_len // NW
    n_chunks = rows_per_w // CHUNK
    mesh = plsc.VectorSubcoreMesh(core_axis_name="c", subcore_axis_name="s")

    @functools.partial(
        pl.kernel,
        out_type=jax.ShapeDtypeStruct((seq_len, b_sz, emb), dtype),
        mesh=mesh,
        scratch_types=[
            pltpu.VMEM((NBUF, CHUNK, emb), dtype),
            pltpu.SemaphoreType.DMA,
            pltpu.SemaphoreType.DMA,
        ],
    )
    def sc_kernel(w_hbm, out_hbm, buf, rsem, wsem):
        wid = lax.axis_index("s") * NC + lax.axis_index("c")
        base = wid * rows_per_w
        descs = [
            pltpu.async_copy(
                w_hbm.at[pl.ds(base, rows_per_w)],
                out_hbm.at[pl.ds(base, rows_per_w), b],
                wsem,
            )
            for b in range(b_sz)
        ]
        for d in descs:
            d.wait()
        return

        def read(c):
            return pltpu.async_copy(
                w_hbm.at[pl.ds(base + c * CHUNK, CHUNK)], buf.at[c % NBUF], rsem
            )

        def writes(c):
            return [
                pltpu.async_copy(
                    buf.at[c % NBUF],
                    out_hbm.at[pl.ds(base + c * CHUNK, CHUNK), b],
                    wsem,
                )
                for b in range(b_sz)
            ]

        rds = {c: None for c in range(n_chunks)}
        wrs = {}
        rds[0] = read(0)
        if n_chunks > 1:
            rds[1] = read(1)
        for c in range(n_chunks):
            rds[c].wait()
            wrs[c] = writes(c)
            if c + NBUF < n_chunks:
                for d in wrs[c]:
                    d.wait()
                rds[c + NBUF] = read(c + NBUF)
        for c in range(max(0, n_chunks - NBUF), n_chunks):
            for d in wrs[c]:
                d.wait()

    return sc_kernel


def kernel(inputs, W):
    seq_len, b_sz = inputs.shape
    emb = W.shape[1]
    return _make_sc_kernel(seq_len, b_sz, emb, W.dtype)(W[:seq_len])


# speedup vs baseline: 42.6662x; 42.6662x over previous
import functools

import jax
import jax.numpy as jnp
from jax import lax
from jax.experimental import pallas as pl
from jax.experimental.pallas import tpu as pltpu
from jax.experimental.pallas import tpu_sc as plsc

NC = 2
NS = 16
NW = NC * NS
CHUNK = 32


NBUF = 3


def _make_sc_kernel(seq_len, b_sz, emb, dtype):
    rows_per_w = seq_len // NW
    n_chunks = rows_per_w // CHUNK
    mesh = plsc.VectorSubcoreMesh(core_axis_name="c", subcore_axis_name="s")

    @functools.partial(
        pl.kernel,
        out_type=jax.ShapeDtypeStruct((seq_len, b_sz, emb), dtype),
        mesh=mesh,
        scratch_types=[
            pltpu.VMEM((NBUF, CHUNK, emb), dtype),
            pltpu.SemaphoreType.DMA,
            pltpu.SemaphoreType.DMA,
            pltpu.SemaphoreType.DMA,
        ],
    )
    def sc_kernel(w_hbm, out_hbm, buf, rsem, wsem, w0sem):
        wid = lax.axis_index("s") * NC + lax.axis_index("c")
        base = wid * rows_per_w

        def read(c):
            return pltpu.async_copy(
                w_hbm.at[pl.ds(base + c * CHUNK, CHUNK)], buf.at[c % NBUF], rsem
            )

        def writes(c, sem):
            return [
                pltpu.async_copy(
                    buf.at[c % NBUF],
                    out_hbm.at[pl.ds(base + c * CHUNK, CHUNK), b],
                    sem,
                )
                for b in range(b_sz)
            ]

        n_reused = max(0, n_chunks - NBUF)
        rds = {}
        wrs = {}
        for c in range(min(NBUF, n_chunks)):
            rds[c] = read(c)
        for c in range(n_chunks):
            if c >= NBUF:
                for d in wrs[c - NBUF]:
                    d.wait()
                rds[c] = read(c)
            rds[c].wait()
            wrs[c] = writes(c, w0sem if c < n_reused else wsem)
        for c in range(n_reused, n_chunks):
            for d in wrs[c]:
                d.wait()

    return sc_kernel


def kernel(inputs, W):
    seq_len, b_sz = inputs.shape
    emb = W.shape[1]
    return _make_sc_kernel(seq_len, b_sz, emb, W.dtype)(W[:seq_len])
